# initial kernel scaffold (unmeasured)
import jax
import jax.numpy as jnp
from jax import lax
from jax.experimental import pallas as pl
from jax.experimental.pallas import tpu as pltpu

N_DEV = 8


def _ring(x):
    return jnp.where(x < 4, x, 11 - x).astype(jnp.int32)


def kernel(t, W):
    m, k = t.shape
    _, n = W.shape
    mc = m // N_DEV

    def body(t_ref, w_ref, out_ref, acc_ref, send_sems, recv_sems):
        my_pos = lax.axis_index("i")
        r = _ring(my_pos)
        nxt = _ring((r + 1) % N_DEV)
        prv = _ring((r - 1) % N_DEV)

        barrier_sem = pltpu.get_barrier_semaphore()
        for nbr in (prv, nxt):
            pl.semaphore_signal(
                barrier_sem,
                inc=1,
                device_id=(nbr,),
                device_id_type=pl.DeviceIdType.MESH,
            )
        pl.semaphore_wait(barrier_sem, 2)

        acc_ref[0] = t_ref[pl.ds(r * mc, mc), :]
        for h in range(N_DEV - 1):
            rdma = pltpu.make_async_remote_copy(
                src_ref=acc_ref.at[h],
                dst_ref=acc_ref.at[h + 1],
                send_sem=send_sems.at[h],
                recv_sem=recv_sems.at[h],
                device_id=(nxt,),
                device_id_type=pl.DeviceIdType.MESH,
            )
            rdma.start()
            rdma.wait()
            c = (r - 1 - h) % N_DEV
            acc_ref[h + 1] += t_ref[pl.ds(c * mc, mc), :]

        c_own = (r + 1) % N_DEV
        out_ref[pl.ds(c_own * mc, mc), :] = jnp.dot(
            acc_ref[N_DEV - 1], w_ref[:, :], preferred_element_type=jnp.float32
        )

        for h in range(N_DEV - 1):
            c_send = (c_own - h) % N_DEV
            rdma = pltpu.make_async_remote_copy(
                src_ref=out_ref.at[pl.ds(c_send * mc, mc), :],
                dst_ref=out_ref.at[pl.ds(c_send * mc, mc), :],
                send_sem=send_sems.at[(N_DEV - 1) + h],
                recv_sem=recv_sems.at[(N_DEV - 1) + h],
                device_id=(nxt,),
                device_id_type=pl.DeviceIdType.MESH,
            )
            rdma.start()
            rdma.wait()

    return pl.pallas_call(
        body,
        out_shape=jax.ShapeDtypeStruct((m, n), jnp.float32),
        in_specs=[
            pl.BlockSpec(memory_space=pltpu.VMEM),
            pl.BlockSpec(memory_space=pltpu.VMEM),
        ],
        out_specs=pl.BlockSpec(memory_space=pltpu.VMEM),
        scratch_shapes=[
            pltpu.VMEM((N_DEV, mc, k), jnp.float32),
            pltpu.SemaphoreType.DMA((2 * (N_DEV - 1),)),
            pltpu.SemaphoreType.DMA((2 * (N_DEV - 1),)),
        ],
        compiler_params=pltpu.CompilerParams(collective_id=0),
    )(t, W)


# baseline (device time: 354647 ns/iter reference)
import jax
import jax.numpy as jnp
from jax import lax
from jax.experimental import pallas as pl
from jax.experimental.pallas import tpu as pltpu

N_DEV = 8


def _ring(x):
    return jnp.where(x < 4, x, 11 - x).astype(jnp.int32)


def kernel(t, W):
    m, k = t.shape
    _, n = W.shape
    mc = m // N_DEV

    def body(t_hbm, w_ref, out_hbm, acc_ref, stage_ref, send_sems, recv_sems, local_sems):
        my_pos = lax.axis_index("i")
        r = _ring(my_pos)
        nxt = _ring((r + 1) % N_DEV)
        prv = _ring((r - 1) % N_DEV)

        barrier_sem = pltpu.get_barrier_semaphore()
        for nbr in (prv, nxt):
            pl.semaphore_signal(
                barrier_sem,
                inc=1,
                device_id=(nbr,),
                device_id_type=pl.DeviceIdType.MESH,
            )
        pl.semaphore_wait(barrier_sem, 2)

        def load_chunk(c, dst, sem):
            cp = pltpu.make_async_copy(
                t_hbm.at[pl.ds(c * mc, mc), :], dst, sem
            )
            cp.start()
            return cp

        load_chunk(r, acc_ref.at[0], local_sems.at[0]).wait()
        for h in range(N_DEV - 1):
            c = (r - 1 - h) % N_DEV
            ld = load_chunk(c, stage_ref, local_sems.at[0])
            rdma = pltpu.make_async_remote_copy(
                src_ref=acc_ref.at[h],
                dst_ref=acc_ref.at[h + 1],
                send_sem=send_sems.at[h],
                recv_sem=recv_sems.at[h],
                device_id=(nxt,),
                device_id_type=pl.DeviceIdType.MESH,
            )
            rdma.start()
            ld.wait()
            rdma.wait()
            acc_ref[h + 1] += stage_ref[:, :]

        c_own = (r + 1) % N_DEV
        acc_ref[N_DEV] = jnp.dot(
            acc_ref[N_DEV - 1], w_ref[:, :], preferred_element_type=jnp.float32
        )

        def store_chunk(slot, c, sem):
            cp = pltpu.make_async_copy(
                acc_ref.at[slot], out_hbm.at[pl.ds(c * mc, mc), :], sem
            )
            cp.start()
            return cp

        st = store_chunk(N_DEV, c_own, local_sems.at[1])

        for h in range(N_DEV - 1):
            rdma = pltpu.make_async_remote_copy(
                src_ref=acc_ref.at[N_DEV if h == 0 else h - 1],
                dst_ref=acc_ref.at[h],
                send_sem=send_sems.at[(N_DEV - 1) + h],
                recv_sem=recv_sems.at[(N_DEV - 1) + h],
                device_id=(nxt,),
                device_id_type=pl.DeviceIdType.MESH,
            )
            rdma.start()
            rdma.wait()
            st.wait()
            c_recv = (r - h) % N_DEV
            st = store_chunk(h, c_recv, local_sems.at[1])
        st.wait()

    return pl.pallas_call(
        body,
        out_shape=jax.ShapeDtypeStruct((m, n), jnp.float32),
        in_specs=[
            pl.BlockSpec(memory_space=pl.ANY),
            pl.BlockSpec(memory_space=pltpu.VMEM),
        ],
        out_specs=pl.BlockSpec(memory_space=pl.ANY),
        scratch_shapes=[
            pltpu.VMEM((N_DEV + 1, mc, n), jnp.float32),
            pltpu.VMEM((mc, k), jnp.float32),
            pltpu.SemaphoreType.DMA((2 * (N_DEV - 1),)),
            pltpu.SemaphoreType.DMA((2 * (N_DEV - 1),)),
            pltpu.SemaphoreType.DMA((2,)),
        ],
        compiler_params=pltpu.CompilerParams(collective_id=0),
    )(t, W)


# device time: 145887 ns/iter; 2.4310x vs baseline; 2.4310x over previous
import jax
import jax.numpy as jnp
from jax import lax
from jax.experimental import pallas as pl
from jax.experimental.pallas import tpu as pltpu

N_DEV = 8
PARTS = ((0, 1536), (1536, 1536), (3072, 1024))
DIMS = ((0, 1, 2), (1, 2, 0), (2, 0, 1))


def kernel(t, W):
    m, k = t.shape
    _, n = W.shape

    def body(t_hbm, w_ref, out_hbm,
             wk0, wk1, wk2, ra0, ra1, ra2, rb0, rb1, rb2, rc0, rc1, rc2,
             send_sems, recv_sems, ld_sems, st_sems, credit_sems):
        works = (wk0, wk1, wk2)
        recv0 = (ra0, ra1, ra2)
        recv1 = (rb0, rb1, rb2)
        recv2 = (rc0, rc1, rc2)

        p = lax.axis_index("i")
        bz = p // 4
        q = p - 4 * bz
        by = q // 2
        bx = ((q == 1) | (q == 2)).astype(jnp.int32)
        bits = (bx, by, bz)
        nbrs = (
            4 * bz + q + 1 - 2 * (q - 2 * by),
            4 * bz + 3 - q,
            (p + 4) % N_DEV,
        )

        def rdma(src, dst, i, s, d):
            return pltpu.make_async_remote_copy(
                src_ref=src,
                dst_ref=dst,
                send_sem=send_sems.at[i * 9 + s],
                recv_sem=recv_sems.at[i * 9 + s],
                device_id=(nbrs[d],),
                device_id_type=pl.DeviceIdType.MESH,
            )

        barrier_sem = pltpu.get_barrier_semaphore()
        for d in range(3):
            pl.semaphore_signal(
                barrier_sem, inc=1, device_id=(nbrs[d],),
                device_id_type=pl.DeviceIdType.MESH,
            )
        pl.semaphore_wait(barrier_sem, 3)

        ops = []
        for i in range(3):
            ps, S = PARTS[i]
            h = S // 2
            b = bits[DIMS[i][0]]
            ld = pltpu.make_async_copy(
                t_hbm.at[pl.ds(ps + b * h, h), :], works[i], ld_sems.at[i]
            )
            ld.start()
            rd = rdma(
                t_hbm.at[pl.ds(ps + (1 - b) * h, h), :], recv0[i],
                i, 0, DIMS[i][0],
            )
            rd.start()
            ops.append((ld, rd))
        for i in range(3):
            ld, rd = ops[i]
            ld.wait()
            rd.wait()
            works[i][:, :] += recv0[i][:, :]

        ops = []
        for i in range(3):
            _, S = PARTS[i]
            h = S // 4
            b = bits[DIMS[i][1]]
            rd = rdma(
                works[i].at[pl.ds((1 - b) * h, h), :], recv1[i],
                i, 1, DIMS[i][1],
            )
            rd.start()
            ops.append(rd)
        for i in range(3):
            _, S = PARTS[i]
            h = S // 4
            b = bits[DIMS[i][1]]
            ops[i].wait()
            recv1[i][:, :] += works[i][pl.ds(b * h, h), :]

        ops = []
        for i in range(3):
            _, S = PARTS[i]
            h = S // 8
            b = bits[DIMS[i][2]]
            rd = rdma(
                recv1[i].at[pl.ds((1 - b) * h, h), :], recv2[i],
                i, 2, DIMS[i][2],
            )
            rd.start()
            ops.append(rd)
        for i in range(3):
            _, S = PARTS[i]
            h = S // 8
            b = bits[DIMS[i][2]]
            ops[i].wait()
            recv2[i][:, :] += recv1[i][pl.ds(b * h, h), :]

        for i in range(3):
            _, S = PARTS[i]
            s8 = S // 8
            recv0[i][pl.ds(0, s8), :] = jnp.dot(
                recv2[i][:, :], w_ref[:, :],
                preferred_element_type=jnp.float32,
            )

        sts = []
        for i in range(3):
            ps, S = PARTS[i]
            s2, s4, s8 = S // 2, S // 4, S // 8
            d0, d1, d2 = DIMS[i]
            off3 = bits[d0] * s2 + bits[d1] * s4 + bits[d2] * s8
            st = pltpu.make_async_copy(
                recv0[i].at[pl.ds(0, s8), :],
                out_hbm.at[pl.ds(ps + off3, s8), :],
                st_sems.at[i * 4 + 0],
            )
            st.start()
            sts.append(st)

        for d in range(3):
            pl.semaphore_signal(
                credit_sems.at[d], inc=1, device_id=(nbrs[d],),
                device_id_type=pl.DeviceIdType.MESH,
            )
        for d in range(3):
            pl.semaphore_wait(credit_sems.at[d], 1)

        ops = []
        for i in range(3):
            _, S = PARTS[i]
            s8 = S // 8
            rd = rdma(
                recv0[i].at[pl.ds(0, s8), :],
                recv0[i].at[pl.ds(s8, s8), :],
                i, 3, DIMS[i][2],
            )
            rd.start()
            ops.append(rd)
        for i in range(3):
            ops[i].wait()
        for i in range(3):
            ps, S = PARTS[i]
            s2, s4, s8 = S // 2, S // 4, S // 8
            d0, d1, d2 = DIMS[i]
            off3x = bits[d0] * s2 + bits[d1] * s4 + (1 - bits[d2]) * s8
            st = pltpu.make_async_copy(
                recv0[i].at[pl.ds(s8, s8), :],
                out_hbm.at[pl.ds(ps + off3x, s8), :],
                st_sems.at[i * 4 + 1],
            )
            st.start()
            sts.append(st)

        ops = []
        for i in range(3):
            _, S = PARTS[i]
            s8 = S // 8
            b2 = bits[DIMS[i][2]]
            rd_lo = rdma(
                recv0[i].at[pl.ds(b2 * s8, s8), :],
                recv1[i].at[pl.ds(0, s8), :],
                i, 4, DIMS[i][1],
            )
            rd_lo.start()
            rd_hi = rdma(
                recv0[i].at[pl.ds((1 - b2) * s8, s8), :],
                recv1[i].at[pl.ds(s8, s8), :],
                i, 5, DIMS[i][1],
            )
            rd_hi.start()
            ops.append((rd_lo, rd_hi))
        for i in range(3):
            ops[i][0].wait()
            ops[i][1].wait()
        for i in range(3):
            ps, S = PARTS[i]
            s2, s4 = S // 2, S // 4
            d0, d1, _ = DIMS[i]
            off2x = bits[d0] * s2 + (1 - bits[d1]) * s4
            st = pltpu.make_async_copy(
                recv1[i],
                out_hbm.at[pl.ds(ps + off2x, s4), :],
                st_sems.at[i * 4 + 2],
            )
            st.start()
            sts.append(st)

        ops = []
        for i in range(3):
            _, S = PARTS[i]
            s4, s8 = S // 4, S // 8
            b1 = bits[DIMS[i][1]]
            b2 = bits[DIMS[i][2]]
            qoff = b1 * s4
            rd_lo = rdma(
                recv0[i].at[pl.ds(b2 * s8, s8), :],
                works[i].at[pl.ds(qoff, s8), :],
                i, 6, DIMS[i][0],
            )
            rd_lo.start()
            rd_hi = rdma(
                recv0[i].at[pl.ds((1 - b2) * s8, s8), :],
                works[i].at[pl.ds(qoff + s8, s8), :],
                i, 7, DIMS[i][0],
            )
            rd_hi.start()
            rd_q = rdma(
                recv1[i],
                works[i].at[pl.ds((1 - b1) * s4, s4), :],
                i, 8, DIMS[i][0],
            )
            rd_q.start()
            ops.append((rd_lo, rd_hi, rd_q))
        for i in range(3):
            for rd in ops[i]:
                rd.wait()
        for i in range(3):
            ps, S = PARTS[i]
            s2 = S // 2
            d0 = DIMS[i][0]
            off1x = (1 - bits[d0]) * s2
            st = pltpu.make_async_copy(
                works[i],
                out_hbm.at[pl.ds(ps + off1x, s2), :],
                st_sems.at[i * 4 + 3],
            )
            st.start()
            sts.append(st)

        for st in sts:
            st.wait()

    _, parts_rows = zip(*PARTS)
    scratch = []
    for rows in parts_rows:
        scratch.append(pltpu.VMEM((rows // 2, n), jnp.float32))
    for rows in parts_rows:
        scratch.append(pltpu.VMEM((rows // 2, n), jnp.float32))
    for rows in parts_rows:
        scratch.append(pltpu.VMEM((rows // 4, n), jnp.float32))
    for rows in parts_rows:
        scratch.append(pltpu.VMEM((rows // 8, n), jnp.float32))
    scratch += [
        pltpu.SemaphoreType.DMA((27,)),
        pltpu.SemaphoreType.DMA((27,)),
        pltpu.SemaphoreType.DMA((3,)),
        pltpu.SemaphoreType.DMA((12,)),
        pltpu.SemaphoreType.REGULAR((3,)),
    ]

    return pl.pallas_call(
        body,
        out_shape=jax.ShapeDtypeStruct((m, n), jnp.float32),
        in_specs=[
            pl.BlockSpec(memory_space=pl.ANY),
            pl.BlockSpec(memory_space=pltpu.VMEM),
        ],
        out_specs=pl.BlockSpec(memory_space=pl.ANY),
        scratch_shapes=scratch,
        compiler_params=pltpu.CompilerParams(collective_id=0),
    )(t, W)


# device time: 136922 ns/iter; 2.5901x vs baseline; 1.0655x over previous
import jax
import jax.numpy as jnp
from jax import lax
from jax.experimental import pallas as pl
from jax.experimental.pallas import tpu as pltpu

N_DEV = 8
PARTS = ((0, 1344), (1344, 1344), (2688, 1408))
DIMS = ((0, 1, 2), (1, 2, 0), (2, 0, 1))


def kernel(t, W):
    m, k = t.shape
    _, n = W.shape

    def body(t_hbm, w_ref, out_hbm,
             wk0, wk1, wk2, ra0, ra1, ra2, rb0, rb1, rb2, rc0, rc1, rc2,
             send_sems, recv_sems, ld_sems, st_sems, credit_sems):
        works = (wk0, wk1, wk2)
        recv0 = (ra0, ra1, ra2)
        recv1 = (rb0, rb1, rb2)
        recv2 = (rc0, rc1, rc2)

        p = lax.axis_index("i")
        bz = p // 4
        q = p - 4 * bz
        by = q // 2
        bx = ((q == 1) | (q == 2)).astype(jnp.int32)
        bits = (bx, by, bz)
        nbrs = (
            4 * bz + q + 1 - 2 * (q - 2 * by),
            4 * bz + 3 - q,
            (p + 4) % N_DEV,
        )

        def rdma(src, dst, i, s, d):
            return pltpu.make_async_remote_copy(
                src_ref=src,
                dst_ref=dst,
                send_sem=send_sems.at[i * 9 + s],
                recv_sem=recv_sems.at[i * 9 + s],
                device_id=(nbrs[d],),
                device_id_type=pl.DeviceIdType.MESH,
            )

        barrier_sem = pltpu.get_barrier_semaphore()
        for d in range(3):
            pl.semaphore_signal(
                barrier_sem, inc=1, device_id=(nbrs[d],),
                device_id_type=pl.DeviceIdType.MESH,
            )
        pl.semaphore_wait(barrier_sem, 3)

        ops = []
        for i in range(3):
            ps, S = PARTS[i]
            h = S // 2
            b = bits[DIMS[i][0]]
            ld = pltpu.make_async_copy(
                t_hbm.at[pl.ds(ps + b * h, h), :], works[i], ld_sems.at[i]
            )
            ld.start()
            rd = rdma(
                t_hbm.at[pl.ds(ps + (1 - b) * h, h), :], recv0[i],
                i, 0, DIMS[i][0],
            )
            rd.start()
            ops.append((ld, rd))
        for i in range(3):
            ld, rd = ops[i]
            ld.wait()
            rd.wait()
            works[i][:, :] += recv0[i][:, :]

        ops = []
        for i in range(3):
            _, S = PARTS[i]
            h = S // 4
            b = bits[DIMS[i][1]]
            rd = rdma(
                works[i].at[pl.ds((1 - b) * h, h), :], recv1[i],
                i, 1, DIMS[i][1],
            )
            rd.start()
            ops.append(rd)
        for i in range(3):
            _, S = PARTS[i]
            h = S // 4
            b = bits[DIMS[i][1]]
            ops[i].wait()
            recv1[i][:, :] += works[i][pl.ds(b * h, h), :]

        ops = []
        for i in range(3):
            _, S = PARTS[i]
            h = S // 8
            b = bits[DIMS[i][2]]
            rd = rdma(
                recv1[i].at[pl.ds((1 - b) * h, h), :], recv2[i],
                i, 2, DIMS[i][2],
            )
            rd.start()
            ops.append(rd)
        for i in range(3):
            _, S = PARTS[i]
            h = S // 8
            b = bits[DIMS[i][2]]
            ops[i].wait()
            recv2[i][:, :] += recv1[i][pl.ds(b * h, h), :]

        for d in range(3):
            pl.semaphore_signal(
                credit_sems.at[d], inc=1, device_id=(nbrs[d],),
                device_id_type=pl.DeviceIdType.MESH,
            )

        sts = []
        ops = []
        credited = set()
        for i in range(3):
            ps, S = PARTS[i]
            s2, s4, s8 = S // 2, S // 4, S // 8
            d0, d1, d2 = DIMS[i]
            recv0[i][pl.ds(0, s8), :] = jnp.dot(
                recv2[i][:, :], w_ref[:, :],
                preferred_element_type=jnp.float32,
            )
            off3 = bits[d0] * s2 + bits[d1] * s4 + bits[d2] * s8
            st = pltpu.make_async_copy(
                recv0[i].at[pl.ds(0, s8), :],
                out_hbm.at[pl.ds(ps + off3, s8), :],
                st_sems.at[i * 4 + 0],
            )
            st.start()
            sts.append(st)
            if d2 not in credited:
                pl.semaphore_wait(credit_sems.at[d2], 1)
                credited.add(d2)
            rd = rdma(
                recv0[i].at[pl.ds(0, s8), :],
                recv0[i].at[pl.ds(s8, s8), :],
                i, 3, d2,
            )
            rd.start()
            ops.append(rd)
        for d in range(3):
            if d not in credited:
                pl.semaphore_wait(credit_sems.at[d], 1)
                credited.add(d)
        for i in range(3):
            ops[i].wait()
        for i in range(3):
            ps, S = PARTS[i]
            s2, s4, s8 = S // 2, S // 4, S // 8
            d0, d1, d2 = DIMS[i]
            off3x = bits[d0] * s2 + bits[d1] * s4 + (1 - bits[d2]) * s8
            st = pltpu.make_async_copy(
                recv0[i].at[pl.ds(s8, s8), :],
                out_hbm.at[pl.ds(ps + off3x, s8), :],
                st_sems.at[i * 4 + 1],
            )
            st.start()
            sts.append(st)

        ops = []
        for i in range(3):
            _, S = PARTS[i]
            s8 = S // 8
            b2 = bits[DIMS[i][2]]
            rd_lo = rdma(
                recv0[i].at[pl.ds(b2 * s8, s8), :],
                recv1[i].at[pl.ds(0, s8), :],
                i, 4, DIMS[i][1],
            )
            rd_lo.start()
            rd_hi = rdma(
                recv0[i].at[pl.ds((1 - b2) * s8, s8), :],
                recv1[i].at[pl.ds(s8, s8), :],
                i, 5, DIMS[i][1],
            )
            rd_hi.start()
            ops.append((rd_lo, rd_hi))
        for i in range(3):
            ops[i][0].wait()
            ops[i][1].wait()
        for i in range(3):
            ps, S = PARTS[i]
            s2, s4 = S // 2, S // 4
            d0, d1, _ = DIMS[i]
            off2x = bits[d0] * s2 + (1 - bits[d1]) * s4
            st = pltpu.make_async_copy(
                recv1[i],
                out_hbm.at[pl.ds(ps + off2x, s4), :],
                st_sems.at[i * 4 + 2],
            )
            st.start()
            sts.append(st)

        ops = []
        for i in range(3):
            _, S = PARTS[i]
            s4, s8 = S // 4, S // 8
            b1 = bits[DIMS[i][1]]
            b2 = bits[DIMS[i][2]]
            qoff = b1 * s4
            rd_lo = rdma(
                recv0[i].at[pl.ds(b2 * s8, s8), :],
                works[i].at[pl.ds(qoff, s8), :],
                i, 6, DIMS[i][0],
            )
            rd_lo.start()
            rd_hi = rdma(
                recv0[i].at[pl.ds((1 - b2) * s8, s8), :],
                works[i].at[pl.ds(qoff + s8, s8), :],
                i, 7, DIMS[i][0],
            )
            rd_hi.start()
            rd_q = rdma(
                recv1[i],
                works[i].at[pl.ds((1 - b1) * s4, s4), :],
                i, 8, DIMS[i][0],
            )
            rd_q.start()
            ops.append((rd_lo, rd_hi, rd_q))
        for i in range(3):
            for rd in ops[i]:
                rd.wait()
        for i in range(3):
            ps, S = PARTS[i]
            s2 = S // 2
            d0 = DIMS[i][0]
            off1x = (1 - bits[d0]) * s2
            st = pltpu.make_async_copy(
                works[i],
                out_hbm.at[pl.ds(ps + off1x, s2), :],
                st_sems.at[i * 4 + 3],
            )
            st.start()
            sts.append(st)

        for st in sts:
            st.wait()

    _, parts_rows = zip(*PARTS)
    scratch = []
    for rows in parts_rows:
        scratch.append(pltpu.VMEM((rows // 2, n), jnp.float32))
    for rows in parts_rows:
        scratch.append(pltpu.VMEM((rows // 2, n), jnp.float32))
    for rows in parts_rows:
        scratch.append(pltpu.VMEM((rows // 4, n), jnp.float32))
    for rows in parts_rows:
        scratch.append(pltpu.VMEM((rows // 8, n), jnp.float32))
    scratch += [
        pltpu.SemaphoreType.DMA((27,)),
        pltpu.SemaphoreType.DMA((27,)),
        pltpu.SemaphoreType.DMA((3,)),
        pltpu.SemaphoreType.DMA((12,)),
        pltpu.SemaphoreType.REGULAR((3,)),
    ]

    return pl.pallas_call(
        body,
        out_shape=jax.ShapeDtypeStruct((m, n), jnp.float32),
        in_specs=[
            pl.BlockSpec(memory_space=pl.ANY),
            pl.BlockSpec(memory_space=pltpu.VMEM),
        ],
        out_specs=pl.BlockSpec(memory_space=pl.ANY),
        scratch_shapes=scratch,
        compiler_params=pltpu.CompilerParams(collective_id=0),
    )(t, W)


# device time: 127311 ns/iter; 2.7857x vs baseline; 1.0755x over previous
import jax
import jax.numpy as jnp
from jax import lax
from jax.experimental import pallas as pl
from jax.experimental.pallas import tpu as pltpu

N_DEV = 8
PARTS = ((0, 1344), (1344, 1344), (2688, 1408))
DIMS = ((0, 1, 2), (1, 2, 0), (2, 0, 1))
NSEM = 11


def kernel(t, W):
    m, k = t.shape
    _, n = W.shape

    def body(t_hbm, w_ref, out_hbm,
             wk0, wk1, wk2, ra0, ra1, ra2, rb0, rb1, rb2, rc0, rc1, rc2,
             send_sems, recv_sems, ld_sems, st_sems, credit_sems):
        works = (wk0, wk1, wk2)
        recv0 = (ra0, ra1, ra2)
        recv1 = (rb0, rb1, rb2)
        recv2 = (rc0, rc1, rc2)

        p = lax.axis_index("i")
        bz = p // 4
        q = p - 4 * bz
        by = q // 2
        bx = ((q == 1) | (q == 2)).astype(jnp.int32)
        bits = (bx, by, bz)
        nbrs = (
            4 * bz + q + 1 - 2 * (q - 2 * by),
            4 * bz + 3 - q,
            (p + 4) % N_DEV,
        )

        B = [[bits[d] for d in DIMS[i]] for i in range(3)]

        def rdma(src, dst, i, s, d):
            return pltpu.make_async_remote_copy(
                src_ref=src,
                dst_ref=dst,
                send_sem=send_sems.at[i * NSEM + s],
                recv_sem=recv_sems.at[i * NSEM + s],
                device_id=(nbrs[d],),
                device_id_type=pl.DeviceIdType.MESH,
            )

        barrier_sem = pltpu.get_barrier_semaphore()
        for d in range(3):
            pl.semaphore_signal(
                barrier_sem, inc=1, device_id=(nbrs[d],),
                device_id_type=pl.DeviceIdType.MESH,
            )
        pl.semaphore_wait(barrier_sem, 3)

        A0, B0, A1, B1, A2, LD = [], [], [], [], [], []
        for i in range(3):
            ps, S = PARTS[i]
            s2, s4, s8 = S // 2, S // 4, S // 8
            b0, b1, b2 = B[i]
            ld = pltpu.make_async_copy(
                t_hbm.at[pl.ds(ps + b0 * s2, s2), :], works[i], ld_sems.at[i]
            )
            ld.start()
            LD.append(ld)
            send_base = ps + (1 - b0) * s2
            a = rdma(
                t_hbm.at[pl.ds(send_base + (1 - b1) * s4, s4), :],
                recv0[i].at[pl.ds((1 - b1) * s4, s4), :],
                i, 0, DIMS[i][0],
            )
            a.start()
            A0.append(a)
            bb = rdma(
                t_hbm.at[pl.ds(send_base + b1 * s4, s4), :],
                recv0[i].at[pl.ds(b1 * s4, s4), :],
                i, 1, DIMS[i][0],
            )
            bb.start()
            B0.append(bb)

        for i in range(3):
            _, S = PARTS[i]
            s4, s8 = S // 4, S // 8
            b0, b1, b2 = B[i]
            LD[i].wait()
            A0[i].wait()
            uo = (1 - b1) * s4
            works[i][pl.ds(uo, s4), :] += recv0[i][pl.ds(uo, s4), :]
            a = rdma(
                works[i].at[pl.ds(uo + (1 - b2) * s8, s8), :],
                recv1[i].at[pl.ds((1 - b2) * s8, s8), :],
                i, 2, DIMS[i][1],
            )
            a.start()
            A1.append(a)

        for i in range(3):
            _, S = PARTS[i]
            s4, s8 = S // 4, S // 8
            b0, b1, b2 = B[i]
            B0[i].wait()
            works[i][pl.ds(b1 * s4, s4), :] += recv0[i][pl.ds(b1 * s4, s4), :]
            bb = rdma(
                works[i].at[pl.ds((1 - b1) * s4 + b2 * s8, s8), :],
                recv1[i].at[pl.ds(b2 * s8, s8), :],
                i, 3, DIMS[i][1],
            )
            bb.start()
            B1.append(bb)

        for i in range(3):
            _, S = PARTS[i]
            s4, s8 = S // 4, S // 8
            b0, b1, b2 = B[i]
            A1[i].wait()
            uo = (1 - b2) * s8
            recv1[i][pl.ds(uo, s8), :] += works[i][pl.ds(b1 * s4 + uo, s8), :]
            a = rdma(
                recv1[i].at[pl.ds(uo, s8), :],
                recv2[i],
                i, 4, DIMS[i][2],
            )
            a.start()
            A2.append(a)

        for i in range(3):
            _, S = PARTS[i]
            s4, s8 = S // 4, S // 8
            b0, b1, b2 = B[i]
            B1[i].wait()
            recv1[i][pl.ds(b2 * s8, s8), :] += works[i][
                pl.ds(b1 * s4 + b2 * s8, s8), :
            ]

        for i in range(3):
            _, S = PARTS[i]
            s8 = S // 8
            b2 = B[i][2]
            A2[i].wait()
            recv2[i][:, :] += recv1[i][pl.ds(b2 * s8, s8), :]

        for d in range(3):
            pl.semaphore_signal(
                credit_sems.at[d], inc=1, device_id=(nbrs[d],),
                device_id_type=pl.DeviceIdType.MESH,
            )

        sts = []
        for i in range(3):
            ps, S = PARTS[i]
            s2, s4, s8 = S // 2, S // 4, S // 8
            b0, b1, b2 = B[i]
            recv0[i][pl.ds(0, s8), :] = jnp.dot(
                recv2[i][:, :], w_ref[:, :],
                preferred_element_type=jnp.float32,
            )
            off3 = b0 * s2 + b1 * s4 + b2 * s8
            st = pltpu.make_async_copy(
                recv0[i].at[pl.ds(0, s8), :],
                out_hbm.at[pl.ds(ps + off3, s8), :],
                st_sems.at[i * 4 + 0],
            )
            st.start()
            sts.append(st)

        for d in range(3):
            pl.semaphore_wait(credit_sems.at[d], 1)

        AG0, AG1Y, AG2Y = [], [], []
        for i in range(3):
            _, S = PARTS[i]
            s4, s8 = S // 4, S // 8
            b0, b1, b2 = B[i]
            y = recv0[i].at[pl.ds(0, s8), :]
            rd = rdma(y, recv0[i].at[pl.ds(s8, s8), :], i, 5, DIMS[i][2])
            rd.start()
            AG0.append(rd)
            rd = rdma(y, recv1[i].at[pl.ds(b2 * s8, s8), :], i, 6, DIMS[i][1])
            rd.start()
            AG1Y.append(rd)
            rd = rdma(
                y, works[i].at[pl.ds(b1 * s4 + b2 * s8, s8), :],
                i, 7, DIMS[i][0],
            )
            rd.start()
            AG2Y.append(rd)

        AG1P, AG2P = [], []
        for i in range(3):
            ps, S = PARTS[i]
            s2, s4, s8 = S // 2, S // 4, S // 8
            b0, b1, b2 = B[i]
            AG0[i].wait()
            p0 = recv0[i].at[pl.ds(s8, s8), :]
            off3x = b0 * s2 + b1 * s4 + (1 - b2) * s8
            st = pltpu.make_async_copy(
                p0, out_hbm.at[pl.ds(ps + off3x, s8), :],
                st_sems.at[i * 4 + 1],
            )
            st.start()
            sts.append(st)
            rd = rdma(
                p0, recv1[i].at[pl.ds((1 - b2) * s8, s8), :],
                i, 8, DIMS[i][1],
            )
            rd.start()
            AG1P.append(rd)
            rd = rdma(
                p0, works[i].at[pl.ds(b1 * s4 + (1 - b2) * s8, s8), :],
                i, 9, DIMS[i][0],
            )
            rd.start()
            AG2P.append(rd)

        AG2Q = []
        for i in range(3):
            ps, S = PARTS[i]
            s2, s4 = S // 2, S // 4
            b0, b1, b2 = B[i]
            AG1Y[i].wait()
            AG1P[i].wait()
            off2x = b0 * s2 + (1 - b1) * s4
            st = pltpu.make_async_copy(
                recv1[i], out_hbm.at[pl.ds(ps + off2x, s4), :],
                st_sems.at[i * 4 + 2],
            )
            st.start()
            sts.append(st)
            rd = rdma(
                recv1[i], works[i].at[pl.ds((1 - b1) * s4, s4), :],
                i, 10, DIMS[i][0],
            )
            rd.start()
            AG2Q.append(rd)

        for i in range(3):
            ps, S = PARTS[i]
            s2 = S // 2
            b0 = B[i][0]
            AG2Y[i].wait()
            AG2P[i].wait()
            AG2Q[i].wait()
            st = pltpu.make_async_copy(
                works[i], out_hbm.at[pl.ds(ps + (1 - b0) * s2, s2), :],
                st_sems.at[i * 4 + 3],
            )
            st.start()
            sts.append(st)

        for st in sts:
            st.wait()

    _, parts_rows = zip(*PARTS)
    scratch = []
    for rows in parts_rows:
        scratch.append(pltpu.VMEM((rows // 2, n), jnp.float32))
    for rows in parts_rows:
        scratch.append(pltpu.VMEM((rows // 2, n), jnp.float32))
    for rows in parts_rows:
        scratch.append(pltpu.VMEM((rows // 4, n), jnp.float32))
    for rows in parts_rows:
        scratch.append(pltpu.VMEM((rows // 8, n), jnp.float32))
    scratch += [
        pltpu.SemaphoreType.DMA((3 * NSEM,)),
        pltpu.SemaphoreType.DMA((3 * NSEM,)),
        pltpu.SemaphoreType.DMA((3,)),
        pltpu.SemaphoreType.DMA((12,)),
        pltpu.SemaphoreType.REGULAR((3,)),
    ]

    return pl.pallas_call(
        body,
        out_shape=jax.ShapeDtypeStruct((m, n), jnp.float32),
        in_specs=[
            pl.BlockSpec(memory_space=pl.ANY),
            pl.BlockSpec(memory_space=pltpu.VMEM),
        ],
        out_specs=pl.BlockSpec(memory_space=pl.ANY),
        scratch_shapes=scratch,
        compiler_params=pltpu.CompilerParams(collective_id=0),
    )(t, W)


# device time: 76656 ns/iter; 4.6265x vs baseline; 1.6608x over previous
import jax
import jax.numpy as jnp
from jax import lax
from jax.experimental import pallas as pl
from jax.experimental.pallas import tpu as pltpu

N_DEV = 8
PARTS = ((0, 1408), (1408, 1408), (2816, 1280))
DIMS = ((0, 1, 2), (1, 2, 0), (2, 0, 1))
NSEM = 11
BF = jnp.bfloat16
F32 = jnp.float32


def kernel(t, W):
    m, k = t.shape
    _, n = W.shape

    def body(t_hbm, w_ref, out_hbm,
             wk0, wk1, wk2, sb0, sb1, sb2, ra0, ra1, ra2, rb0, rb1, rb2,
             rc0, rc1, rc2,
             send_sems, recv_sems, ld_sems, st_sems, credit_sems):
        works = (wk0, wk1, wk2)
        sbuf = (sb0, sb1, sb2)
        r0b = (ra0, ra1, ra2)
        r1b = (rb0, rb1, rb2)
        r2b = (rc0, rc1, rc2)

        p = lax.axis_index("i")
        bz = p // 4
        q = p - 4 * bz
        by = q // 2
        bx = ((q == 1) | (q == 2)).astype(jnp.int32)
        bits = (bx, by, bz)
        nbrs = (
            4 * bz + q + 1 - 2 * (q - 2 * by),
            4 * bz + 3 - q,
            (p + 4) % N_DEV,
        )
        B = [[bits[d] for d in DIMS[i]] for i in range(3)]

        def rdma(src, dst, i, s, d):
            return pltpu.make_async_remote_copy(
                src_ref=src,
                dst_ref=dst,
                send_sem=send_sems.at[i * NSEM + s],
                recv_sem=recv_sems.at[i * NSEM + s],
                device_id=(nbrs[d],),
                device_id_type=pl.DeviceIdType.MESH,
            )

        barrier_sem = pltpu.get_barrier_semaphore()
        for d in range(3):
            pl.semaphore_signal(
                barrier_sem, inc=1, device_id=(nbrs[d],),
                device_id_type=pl.DeviceIdType.MESH,
            )
        pl.semaphore_wait(barrier_sem, 3)

        A0, B0, A1, B1, A2, LD = [], [], [], [], [], []
        for i in range(3):
            ps, S = PARTS[i]
            s2 = S // 2
            b0 = B[i][0]
            ld = pltpu.make_async_copy(
                t_hbm.at[pl.ds(ps + (1 - b0) * s2, s2), :], works[i],
                ld_sems.at[i],
            )
            ld.start()
            LD.append(ld)
        for i in range(3):
            ps, S = PARTS[i]
            s2, s4 = S // 2, S // 4
            b0, b1, _ = B[i]
            LD[i].wait()
            sbuf[i][:, :] = works[i][:, :].astype(BF)
            a = rdma(
                sbuf[i].at[pl.ds((1 - b1) * s4, s4), :],
                r0b[i].at[pl.ds((1 - b1) * s4, s4), :],
                i, 0, DIMS[i][0],
            )
            a.start()
            A0.append(a)
            bb = rdma(
                sbuf[i].at[pl.ds(b1 * s4, s4), :],
                r0b[i].at[pl.ds(b1 * s4, s4), :],
                i, 1, DIMS[i][0],
            )
            bb.start()
            B0.append(bb)
            ld = pltpu.make_async_copy(
                t_hbm.at[pl.ds(ps + b0 * s2, s2), :], works[i],
                ld_sems.at[i],
            )
            ld.start()
            LD[i] = ld

        for i in range(3):
            _, S = PARTS[i]
            s4, s8 = S // 4, S // 8
            _, b1, b2 = B[i]
            uo = (1 - b1) * s4
            LD[i].wait()
            A0[i].wait()
            works[i][pl.ds(uo, s4), :] += r0b[i][pl.ds(uo, s4), :].astype(F32)
            so = uo + (1 - b2) * s8
            sbuf[i][pl.ds(so, s8), :] = works[i][pl.ds(so, s8), :].astype(BF)
            a = rdma(
                sbuf[i].at[pl.ds(so, s8), :],
                r1b[i].at[pl.ds((1 - b2) * s8, s8), :],
                i, 2, DIMS[i][1],
            )
            a.start()
            A1.append(a)

        for i in range(3):
            _, S = PARTS[i]
            s4, s8 = S // 4, S // 8
            _, b1, b2 = B[i]
            uo = (1 - b1) * s4
            B0[i].wait()
            works[i][pl.ds(b1 * s4, s4), :] += r0b[i][
                pl.ds(b1 * s4, s4), :
            ].astype(F32)
            so = uo + b2 * s8
            sbuf[i][pl.ds(so, s8), :] = works[i][pl.ds(so, s8), :].astype(BF)
            bb = rdma(
                sbuf[i].at[pl.ds(so, s8), :],
                r1b[i].at[pl.ds(b2 * s8, s8), :],
                i, 3, DIMS[i][1],
            )
            bb.start()
            B1.append(bb)

        for i in range(3):
            _, S = PARTS[i]
            s4, s8 = S // 4, S // 8
            _, b1, b2 = B[i]
            uo = (1 - b1) * s4
            rel = (1 - b2) * s8
            A1[i].wait()
            works[i][pl.ds(uo + rel, s8), :] = (
                r1b[i][pl.ds(rel, s8), :].astype(F32)
                + works[i][pl.ds(b1 * s4 + rel, s8), :]
            )
            sbuf[i][pl.ds(uo + rel, s8), :] = works[i][
                pl.ds(uo + rel, s8), :
            ].astype(BF)
            a = rdma(
                sbuf[i].at[pl.ds(uo + rel, s8), :],
                r2b[i],
                i, 4, DIMS[i][2],
            )
            a.start()
            A2.append(a)

        for i in range(3):
            _, S = PARTS[i]
            s4, s8 = S // 4, S // 8
            _, b1, b2 = B[i]
            uo = (1 - b1) * s4
            rel = b2 * s8
            B1[i].wait()
            works[i][pl.ds(uo + rel, s8), :] = (
                r1b[i][pl.ds(rel, s8), :].astype(F32)
                + works[i][pl.ds(b1 * s4 + rel, s8), :]
            )

        for i in range(3):
            _, S = PARTS[i]
            s4, s8 = S // 4, S // 8
            _, b1, b2 = B[i]
            A2[i].wait()
            works[i][pl.ds((1 - b1) * s4 + b2 * s8, s8), :] += r2b[i][
                :, :
            ].astype(F32)

        for d in range(3):
            pl.semaphore_signal(
                credit_sems.at[d], inc=1, device_id=(nbrs[d],),
                device_id_type=pl.DeviceIdType.MESH,
            )

        sts = []
        for i in range(3):
            ps, S = PARTS[i]
            s2, s4, s8 = S // 2, S // 4, S // 8
            b0, b1, b2 = B[i]
            fin = works[i][pl.ds((1 - b1) * s4 + b2 * s8, s8), :]
            y = jnp.dot(fin, w_ref[:, :], preferred_element_type=F32)
            works[i][pl.ds(0, s8), :] = y
            r0b[i][pl.ds(0, s8), :] = y.astype(BF)
            off3 = b0 * s2 + b1 * s4 + b2 * s8
            st = pltpu.make_async_copy(
                works[i].at[pl.ds(0, s8), :],
                out_hbm.at[pl.ds(ps + off3, s8), :],
                st_sems.at[i * 4 + 0],
            )
            st.start()
            sts.append(st)

        for d in range(3):
            pl.semaphore_wait(credit_sems.at[d], 1)

        AG0, AG1Y, AG2Y = [], [], []
        for i in range(3):
            _, S = PARTS[i]
            s4, s8 = S // 4, S // 8
            _, b1, b2 = B[i]
            y = r0b[i].at[pl.ds(0, s8), :]
            rd = rdma(y, r0b[i].at[pl.ds(s8, s8), :], i, 5, DIMS[i][2])
            rd.start()
            AG0.append(rd)
            rd = rdma(y, r1b[i].at[pl.ds(b2 * s8, s8), :], i, 6, DIMS[i][1])
            rd.start()
            AG1Y.append(rd)
            rd = rdma(
                y, sbuf[i].at[pl.ds(b1 * s4 + b2 * s8, s8), :],
                i, 7, DIMS[i][0],
            )
            rd.start()
            AG2Y.append(rd)

        AG1P, AG2P = [], []
        for i in range(3):
            ps, S = PARTS[i]
            s2, s4, s8 = S // 2, S // 4, S // 8
            b0, b1, b2 = B[i]
            AG0[i].wait()
            p0 = r0b[i].at[pl.ds(s8, s8), :]
            works[i][pl.ds(s8, s8), :] = r0b[i][pl.ds(s8, s8), :].astype(F32)
            off3x = b0 * s2 + b1 * s4 + (1 - b2) * s8
            st = pltpu.make_async_copy(
                works[i].at[pl.ds(s8, s8), :],
                out_hbm.at[pl.ds(ps + off3x, s8), :],
                st_sems.at[i * 4 + 1],
            )
            st.start()
            sts.append(st)
            rd = rdma(
                p0, r1b[i].at[pl.ds((1 - b2) * s8, s8), :],
                i, 8, DIMS[i][1],
            )
            rd.start()
            AG1P.append(rd)
            rd = rdma(
                p0, sbuf[i].at[pl.ds(b1 * s4 + (1 - b2) * s8, s8), :],
                i, 9, DIMS[i][0],
            )
            rd.start()
            AG2P.append(rd)

        AG2Q = []
        for i in range(3):
            ps, S = PARTS[i]
            s2, s4, s8 = S // 2, S // 4, S // 8
            b0, b1, _ = B[i]
            AG1Y[i].wait()
            AG1P[i].wait()
            works[i][pl.ds(2 * s8, s4), :] = r1b[i][:, :].astype(F32)
            off2x = b0 * s2 + (1 - b1) * s4
            st = pltpu.make_async_copy(
                works[i].at[pl.ds(2 * s8, s4), :],
                out_hbm.at[pl.ds(ps + off2x, s4), :],
                st_sems.at[i * 4 + 2],
            )
            st.start()
            sts.append(st)
            rd = rdma(
                r1b[i], sbuf[i].at[pl.ds((1 - b1) * s4, s4), :],
                i, 10, DIMS[i][0],
            )
            rd.start()
            AG2Q.append(rd)

        for i in range(3):
            AG2Y[i].wait()
            AG2P[i].wait()
            AG2Q[i].wait()
        for st in sts:
            st.wait()
        sts = []
        for i in range(3):
            ps, S = PARTS[i]
            s2 = S // 2
            b0 = B[i][0]
            works[i][:, :] = sbuf[i][:, :].astype(F32)
            st = pltpu.make_async_copy(
                works[i],
                out_hbm.at[pl.ds(ps + (1 - b0) * s2, s2), :],
                st_sems.at[i * 4 + 3],
            )
            st.start()
            sts.append(st)
        for st in sts:
            st.wait()

    _, parts_rows = zip(*PARTS)
    scratch = []
    for rows in parts_rows:
        scratch.append(pltpu.VMEM((rows // 2, n), F32))
    for rows in parts_rows:
        scratch.append(pltpu.VMEM((rows // 2, n), BF))
    for rows in parts_rows:
        scratch.append(pltpu.VMEM((rows // 2, n), BF))
    for rows in parts_rows:
        scratch.append(pltpu.VMEM((rows // 4, n), BF))
    for rows in parts_rows:
        scratch.append(pltpu.VMEM((rows // 8, n), BF))
    scratch += [
        pltpu.SemaphoreType.DMA((3 * NSEM,)),
        pltpu.SemaphoreType.DMA((3 * NSEM,)),
        pltpu.SemaphoreType.DMA((3,)),
        pltpu.SemaphoreType.DMA((12,)),
        pltpu.SemaphoreType.REGULAR((3,)),
    ]

    return pl.pallas_call(
        body,
        out_shape=jax.ShapeDtypeStruct((m, n), F32),
        in_specs=[
            pl.BlockSpec(memory_space=pl.ANY),
            pl.BlockSpec(memory_space=pltpu.VMEM),
        ],
        out_specs=pl.BlockSpec(memory_space=pl.ANY),
        scratch_shapes=scratch,
        compiler_params=pltpu.CompilerParams(collective_id=0),
    )(t, W)


# device time: 76565 ns/iter; 4.6320x vs baseline; 1.0012x over previous
import jax
import jax.numpy as jnp
from jax import lax
from jax.experimental import pallas as pl
from jax.experimental.pallas import tpu as pltpu

N_DEV = 8
PARTS = ((0, 1408), (1408, 1408), (2816, 1280))
DIMS = ((0, 1, 2), (1, 2, 0), (2, 0, 1))
NSEM = 11
BF = jnp.bfloat16
F32 = jnp.float32


def kernel(t, W):
    m, k = t.shape
    _, n = W.shape

    def body(t_hbm, w_ref, out_hbm,
             wk0, wk1, wk2, sb0, sb1, sb2, ra0, ra1, ra2, rb0, rb1, rb2,
             rc0, rc1, rc2, w_bf,
             send_sems, recv_sems, ld_sems, st_sems, credit_sems):
        works = (wk0, wk1, wk2)
        sbuf = (sb0, sb1, sb2)
        r0b = (ra0, ra1, ra2)
        r1b = (rb0, rb1, rb2)
        r2b = (rc0, rc1, rc2)

        p = lax.axis_index("i")
        bz = p // 4
        q = p - 4 * bz
        by = q // 2
        bx = ((q == 1) | (q == 2)).astype(jnp.int32)
        bits = (bx, by, bz)
        nbrs = (
            4 * bz + q + 1 - 2 * (q - 2 * by),
            4 * bz + 3 - q,
            (p + 4) % N_DEV,
        )
        B = [[bits[d] for d in DIMS[i]] for i in range(3)]

        def rdma(src, dst, i, s, d):
            return pltpu.make_async_remote_copy(
                src_ref=src,
                dst_ref=dst,
                send_sem=send_sems.at[i * NSEM + s],
                recv_sem=recv_sems.at[i * NSEM + s],
                device_id=(nbrs[d],),
                device_id_type=pl.DeviceIdType.MESH,
            )

        barrier_sem = pltpu.get_barrier_semaphore()
        for d in range(3):
            pl.semaphore_signal(
                barrier_sem, inc=1, device_id=(nbrs[d],),
                device_id_type=pl.DeviceIdType.MESH,
            )
        pl.semaphore_wait(barrier_sem, 3)

        A0, B0, A1, B1, A2, LD = [], [], [], [], [], []
        for i in range(3):
            ps, S = PARTS[i]
            s2 = S // 2
            b0 = B[i][0]
            ld = pltpu.make_async_copy(
                t_hbm.at[pl.ds(ps + (1 - b0) * s2, s2), :], works[i],
                ld_sems.at[i],
            )
            ld.start()
            LD.append(ld)
        for i in range(3):
            ps, S = PARTS[i]
            s2, s4 = S // 2, S // 4
            b0, b1, _ = B[i]
            LD[i].wait()
            uo = (1 - b1) * s4
            sbuf[i][pl.ds(uo, s4), :] = works[i][pl.ds(uo, s4), :].astype(BF)
            a = rdma(
                sbuf[i].at[pl.ds(uo, s4), :],
                r0b[i].at[pl.ds(uo, s4), :],
                i, 0, DIMS[i][0],
            )
            a.start()
            A0.append(a)
            sbuf[i][pl.ds(b1 * s4, s4), :] = works[i][
                pl.ds(b1 * s4, s4), :
            ].astype(BF)
            bb = rdma(
                sbuf[i].at[pl.ds(b1 * s4, s4), :],
                r0b[i].at[pl.ds(b1 * s4, s4), :],
                i, 1, DIMS[i][0],
            )
            bb.start()
            B0.append(bb)
            ld = pltpu.make_async_copy(
                t_hbm.at[pl.ds(ps + b0 * s2, s2), :], works[i],
                ld_sems.at[i],
            )
            ld.start()
            LD[i] = ld

        w_bf[:, :] = w_ref[:, :].astype(BF)

        for i in range(3):
            _, S = PARTS[i]
            s4, s8 = S // 4, S // 8
            _, b1, b2 = B[i]
            uo = (1 - b1) * s4
            LD[i].wait()
            A0[i].wait()
            works[i][pl.ds(uo, s4), :] += r0b[i][pl.ds(uo, s4), :].astype(F32)
            so = uo + (1 - b2) * s8
            sbuf[i][pl.ds(so, s8), :] = works[i][pl.ds(so, s8), :].astype(BF)
            a = rdma(
                sbuf[i].at[pl.ds(so, s8), :],
                r1b[i].at[pl.ds((1 - b2) * s8, s8), :],
                i, 2, DIMS[i][1],
            )
            a.start()
            A1.append(a)

        for i in range(3):
            _, S = PARTS[i]
            s4, s8 = S // 4, S // 8
            _, b1, b2 = B[i]
            uo = (1 - b1) * s4
            B0[i].wait()
            works[i][pl.ds(b1 * s4, s4), :] += r0b[i][
                pl.ds(b1 * s4, s4), :
            ].astype(F32)
            so = uo + b2 * s8
            sbuf[i][pl.ds(so, s8), :] = works[i][pl.ds(so, s8), :].astype(BF)
            bb = rdma(
                sbuf[i].at[pl.ds(so, s8), :],
                r1b[i].at[pl.ds(b2 * s8, s8), :],
                i, 3, DIMS[i][1],
            )
            bb.start()
            B1.append(bb)

        for i in range(3):
            _, S = PARTS[i]
            s4, s8 = S // 4, S // 8
            _, b1, b2 = B[i]
            uo = (1 - b1) * s4
            rel = (1 - b2) * s8
            A1[i].wait()
            works[i][pl.ds(uo + rel, s8), :] = (
                r1b[i][pl.ds(rel, s8), :].astype(F32)
                + works[i][pl.ds(b1 * s4 + rel, s8), :]
            )
            sbuf[i][pl.ds(uo + rel, s8), :] = works[i][
                pl.ds(uo + rel, s8), :
            ].astype(BF)
            a = rdma(
                sbuf[i].at[pl.ds(uo + rel, s8), :],
                r2b[i],
                i, 4, DIMS[i][2],
            )
            a.start()
            A2.append(a)

        for i in range(3):
            _, S = PARTS[i]
            s4, s8 = S // 4, S // 8
            _, b1, b2 = B[i]
            uo = (1 - b1) * s4
            rel = b2 * s8
            B1[i].wait()
            works[i][pl.ds(uo + rel, s8), :] = (
                r1b[i][pl.ds(rel, s8), :].astype(F32)
                + works[i][pl.ds(b1 * s4 + rel, s8), :]
            )

        for i in range(3):
            _, S = PARTS[i]
            s4, s8 = S // 4, S // 8
            _, b1, b2 = B[i]
            A2[i].wait()
            works[i][pl.ds((1 - b1) * s4 + b2 * s8, s8), :] += r2b[i][
                :, :
            ].astype(F32)

        for d in range(3):
            pl.semaphore_signal(
                credit_sems.at[d], inc=1, device_id=(nbrs[d],),
                device_id_type=pl.DeviceIdType.MESH,
            )

        sts = []
        for i in range(3):
            ps, S = PARTS[i]
            s2, s4, s8 = S // 2, S // 4, S // 8
            b0, b1, b2 = B[i]
            fin = works[i][pl.ds((1 - b1) * s4 + b2 * s8, s8), :]
            y = jnp.dot(
                fin.astype(BF), w_bf[:, :], preferred_element_type=F32
            )
            works[i][pl.ds(0, s8), :] = y
            r0b[i][pl.ds(0, s8), :] = y.astype(BF)
            off3 = b0 * s2 + b1 * s4 + b2 * s8
            st = pltpu.make_async_copy(
                works[i].at[pl.ds(0, s8), :],
                out_hbm.at[pl.ds(ps + off3, s8), :],
                st_sems.at[i * 4 + 0],
            )
            st.start()
            sts.append(st)

        for d in range(3):
            pl.semaphore_wait(credit_sems.at[d], 1)

        AG0, AG1Y, AG2Y = [], [], []
        for i in range(3):
            _, S = PARTS[i]
            s4, s8 = S // 4, S // 8
            _, b1, b2 = B[i]
            y = r0b[i].at[pl.ds(0, s8), :]
            rd = rdma(y, r0b[i].at[pl.ds(s8, s8), :], i, 5, DIMS[i][2])
            rd.start()
            AG0.append(rd)
            rd = rdma(y, r1b[i].at[pl.ds(b2 * s8, s8), :], i, 6, DIMS[i][1])
            rd.start()
            AG1Y.append(rd)
            rd = rdma(
                y, sbuf[i].at[pl.ds(b1 * s4 + b2 * s8, s8), :],
                i, 7, DIMS[i][0],
            )
            rd.start()
            AG2Y.append(rd)

        AG1P, AG2P = [], []
        for i in range(3):
            ps, S = PARTS[i]
            s2, s4, s8 = S // 2, S // 4, S // 8
            b0, b1, b2 = B[i]
            AG0[i].wait()
            p0 = r0b[i].at[pl.ds(s8, s8), :]
            works[i][pl.ds(s8, s8), :] = r0b[i][pl.ds(s8, s8), :].astype(F32)
            off3x = b0 * s2 + b1 * s4 + (1 - b2) * s8
            st = pltpu.make_async_copy(
                works[i].at[pl.ds(s8, s8), :],
                out_hbm.at[pl.ds(ps + off3x, s8), :],
                st_sems.at[i * 4 + 1],
            )
            st.start()
            sts.append(st)
            rd = rdma(
                p0, r1b[i].at[pl.ds((1 - b2) * s8, s8), :],
                i, 8, DIMS[i][1],
            )
            rd.start()
            AG1P.append(rd)
            rd = rdma(
                p0, sbuf[i].at[pl.ds(b1 * s4 + (1 - b2) * s8, s8), :],
                i, 9, DIMS[i][0],
            )
            rd.start()
            AG2P.append(rd)

        AG2Q = []
        for i in range(3):
            ps, S = PARTS[i]
            s2, s4, s8 = S // 2, S // 4, S // 8
            b0, b1, _ = B[i]
            AG1Y[i].wait()
            AG1P[i].wait()
            works[i][pl.ds(2 * s8, s4), :] = r1b[i][:, :].astype(F32)
            off2x = b0 * s2 + (1 - b1) * s4
            st = pltpu.make_async_copy(
                works[i].at[pl.ds(2 * s8, s4), :],
                out_hbm.at[pl.ds(ps + off2x, s4), :],
                st_sems.at[i * 4 + 2],
            )
            st.start()
            sts.append(st)
            rd = rdma(
                r1b[i], sbuf[i].at[pl.ds((1 - b1) * s4, s4), :],
                i, 10, DIMS[i][0],
            )
            rd.start()
            AG2Q.append(rd)

        for i in range(3):
            AG2Y[i].wait()
            AG2P[i].wait()
            AG2Q[i].wait()
        for st in sts:
            st.wait()
        sts = []
        for i in range(3):
            ps, S = PARTS[i]
            s2 = S // 2
            b0 = B[i][0]
            works[i][:, :] = sbuf[i][:, :].astype(F32)
            st = pltpu.make_async_copy(
                works[i],
                out_hbm.at[pl.ds(ps + (1 - b0) * s2, s2), :],
                st_sems.at[i * 4 + 3],
            )
            st.start()
            sts.append(st)
        for st in sts:
            st.wait()

    _, parts_rows = zip(*PARTS)
    scratch = []
    for rows in parts_rows:
        scratch.append(pltpu.VMEM((rows // 2, n), F32))
    for rows in parts_rows:
        scratch.append(pltpu.VMEM((rows // 2, n), BF))
    for rows in parts_rows:
        scratch.append(pltpu.VMEM((rows // 2, n), BF))
    for rows in parts_rows:
        scratch.append(pltpu.VMEM((rows // 4, n), BF))
    for rows in parts_rows:
        scratch.append(pltpu.VMEM((rows // 8, n), BF))
    scratch.append(pltpu.VMEM((k, n), BF))
    scratch += [
        pltpu.SemaphoreType.DMA((3 * NSEM,)),
        pltpu.SemaphoreType.DMA((3 * NSEM,)),
        pltpu.SemaphoreType.DMA((3,)),
        pltpu.SemaphoreType.DMA((12,)),
        pltpu.SemaphoreType.REGULAR((3,)),
    ]

    return pl.pallas_call(
        body,
        out_shape=jax.ShapeDtypeStruct((m, n), F32),
        in_specs=[
            pl.BlockSpec(memory_space=pl.ANY),
            pl.BlockSpec(memory_space=pltpu.VMEM),
        ],
        out_specs=pl.BlockSpec(memory_space=pl.ANY),
        scratch_shapes=scratch,
        compiler_params=pltpu.CompilerParams(collective_id=0),
    )(t, W)
